# Initial kernel scaffold; baseline (speedup 1.0000x reference)
import jax
import jax.numpy as jnp
from jax.experimental import pallas as pl


def kernel(mk, mv, qk):
    raise NotImplementedError("write your pallas kernel here")


# speedup vs baseline: 35.3494x; 35.3494x over previous
import functools
import math

import jax
import jax.numpy as jnp
from jax import lax
from jax.experimental import pallas as pl
from jax.experimental.pallas import tpu as pltpu
from jax.experimental.pallas import tpu_sc as plsc

CK = 64
CV = 128
HW = 900
M = 36000
TOPK = 50

NPAD = 36096
G = 141
H2 = 9
MROWS = 904
NEG = -1e30

NC, NS, LANES = 2, 16, 16
NW = NC * NS
COLS_PER_W = 29

NTILE = 256


def _tc_body(q_ref, k_ref, v_ref, at_ref, mvt_ref):
    at_ref[...] = lax.dot_general(
        q_ref[...], k_ref[...], (((1,), (0,)), ((), ())),
        preferred_element_type=jnp.float32)
    mvt_ref[...] = v_ref[...].T


_tc_call = pl.pallas_call(
    _tc_body,
    grid=(NPAD // NTILE,),
    in_specs=[
        pl.BlockSpec((MROWS, CK), lambda i: (0, 0)),
        pl.BlockSpec((CK, NTILE), lambda i: (0, i)),
        pl.BlockSpec((CV, NTILE), lambda i: (0, i)),
    ],
    out_specs=[
        pl.BlockSpec((MROWS, NTILE), lambda i: (0, i)),
        pl.BlockSpec((NTILE, CV), lambda i: (i, 0)),
    ],
    out_shape=[
        jax.ShapeDtypeStruct((MROWS, NPAD), jnp.float32),
        jax.ShapeDtypeStruct((NPAD, CV), jnp.float32),
    ],
)


def _lane(vec, lane_scalar):
    idx = lax.broadcast(lane_scalar, (LANES,))
    return jnp.take_along_axis(vec, idx, axis=0, mode="promise_in_bounds")[0]


def _sc_body(at_hbm, mvt_hbm, out_hbm, colbuf, l1, l2, tv, ti, rows,
             orow, sem):
    wid = lax.axis_index("s") * NC + lax.axis_index("c")
    iota = lax.iota(jnp.int32, LANES)
    lane0 = iota == 0
    neg = jnp.full((LANES,), NEG, jnp.float32)

    def do_col(col):
        pltpu.sync_copy(at_hbm.at[pl.ds(col, 1)], colbuf)
        for j in range(M // LANES, NPAD // LANES):
            colbuf[0, pl.ds(j * LANES, LANES)] = neg

        def build_l1(g, _):
            mx = colbuf[0, pl.ds(g * 256, LANES)]
            for k in range(1, 16):
                mx = jnp.maximum(mx, colbuf[0, pl.ds(g * 256 + k * LANES, LANES)])
            l1[pl.ds(g * LANES, LANES)] = mx
            return 0

        lax.fori_loop(0, G, build_l1, 0)
        for j in range(G, H2 * 16):
            l1[pl.ds(j * LANES, LANES)] = neg
        for h in range(H2):
            mx = l1[pl.ds(h * 256, LANES)]
            for j in range(1, 16):
                mx = jnp.maximum(mx, l1[pl.ds(h * 256 + j * LANES, LANES)])
            l2[pl.ds(h * LANES, LANES)] = mx

        for c in range(4):
            tv[pl.ds(c * LANES, LANES)] = neg
            ti[pl.ds(c * LANES, LANES)] = jnp.zeros((LANES,), jnp.int32)

        def extract(t, _):
            glob = l2[pl.ds(0, LANES)]
            for h in range(1, H2):
                glob = jnp.maximum(glob, l2[pl.ds(h * LANES, LANES)])
            m = jnp.max(glob, axis=0)
            mvec = lax.broadcast(m, (LANES,))
            lsel = jnp.where(glob == mvec, iota, 999)
            lstar = jnp.min(lsel, axis=0)

            henc = jnp.full((LANES,), 999, jnp.int32)
            for h in range(H2):
                hit = l2[pl.ds(h * LANES, LANES)] == mvec
                henc = jnp.where(hit, jnp.minimum(henc, h), henc)
            hstar = _lane(henc, lstar)

            jenc = jnp.full((LANES,), 999, jnp.int32)
            for j in range(16):
                hit = l1[pl.ds(hstar * 256 + j * LANES, LANES)] == mvec
                jenc = jnp.where(hit, jnp.minimum(jenc, j), jenc)
            gstar = hstar * 16 + _lane(jenc, lstar)

            kenc = jnp.full((LANES,), 999, jnp.int32)
            for k in range(16):
                hit = colbuf[0, pl.ds(gstar * 256 + k * LANES, LANES)] == mvec
                kenc = jnp.where(hit, jnp.minimum(kenc, k), kenc)
            kstar = _lane(kenc, lstar)

            tvec = lax.broadcast(t, (LANES,))
            plsc.store_scatter(tv, [tvec], mvec, mask=lane0)
            plsc.store_scatter(
                ti, [tvec],
                lax.broadcast(gstar * 256 + kstar * 16 + lstar, (LANES,)),
                mask=lane0)

            lvec = lax.broadcast(lstar, (LANES,))
            v = colbuf[0, pl.ds(gstar * 256 + kstar * LANES, LANES)]
            colbuf[0, pl.ds(gstar * 256 + kstar * LANES, LANES)] = (
                jnp.where(iota == lvec, NEG, v))
            mx = colbuf[0, pl.ds(gstar * 256, LANES)]
            for k in range(1, 16):
                mx = jnp.maximum(mx, colbuf[0, pl.ds(gstar * 256 + k * LANES, LANES)])
            l1[pl.ds(gstar * LANES, LANES)] = mx
            mx = l1[pl.ds(hstar * 256, LANES)]
            for j in range(1, 16):
                mx = jnp.maximum(mx, l1[pl.ds(hstar * 256 + j * LANES, LANES)])
            l2[pl.ds(hstar * LANES, LANES)] = mx
            return 0

        lax.fori_loop(0, TOPK, extract, 0)

        v0 = lax.broadcast(tv[pl.ds(0, LANES)][0], (LANES,))
        wvecs = []
        zacc = jnp.zeros((LANES,), jnp.float32)
        for c in range(4):
            e = jnp.exp(tv[pl.ds(c * LANES, LANES)] - v0)
            wvecs.append(e)
            zacc = zacc + e
        zvec = lax.broadcast(jnp.sum(zacc, axis=0), (LANES,))
        wvecs = [w / zvec for w in wvecs]

        pltpu.async_copy(mvt_hbm.at[ti], rows, sem).wait()

        accs = [jnp.zeros((LANES,), jnp.float32) for _ in range(CV // LANES)]
        for k in range(TOPK):
            w = lax.broadcast(wvecs[k // LANES][k % LANES], (LANES,))
            for c in range(CV // LANES):
                accs[c] = accs[c] + w * rows[k, pl.ds(c * LANES, LANES)]
        for c in range(CV // LANES):
            orow[0, pl.ds(c * LANES, LANES)] = accs[c]
        pltpu.sync_copy(orow, out_hbm.at[pl.ds(col, 1)])

    def col_body(i, _):
        col = i * NW + wid

        @pl.when(col < HW)
        def _():
            do_col(col)

        return 0

    lax.fori_loop(0, COLS_PER_W, col_body, 0)


@functools.cache
def _get_sc_call():
    return pl.kernel(
        _sc_body,
        out_type=jax.ShapeDtypeStruct((HW, CV), jnp.float32),
        mesh=plsc.VectorSubcoreMesh(
            core_axis_name="c", subcore_axis_name="s",
            num_cores=NC, num_subcores=NS),
        compiler_params=pltpu.CompilerParams(needs_layout_passes=False),
        scratch_types=[
            pltpu.VMEM((1, NPAD), jnp.float32),
            pltpu.VMEM((H2 * 256,), jnp.float32),
            pltpu.VMEM((H2 * LANES,), jnp.float32),
            pltpu.VMEM((64,), jnp.float32),
            pltpu.VMEM((64,), jnp.int32),
            pltpu.VMEM((64, CV), jnp.float32),
            pltpu.VMEM((1, CV), jnp.float32),
            pltpu.SemaphoreType.DMA,
        ],
    )


def kernel(mk, mv, qk):
    mk_f = mk.reshape(CK, M)
    mv_f = mv.reshape(CV, M)
    qiT = jnp.swapaxes(qk.reshape(CK, HW), 0, 1) / math.sqrt(CK)
    mk_p = jnp.pad(mk_f, ((0, 0), (0, NPAD - M)))
    mv_p = jnp.pad(mv_f, ((0, 0), (0, NPAD - M)))
    qiT_p = jnp.pad(qiT, ((0, MROWS - HW), (0, 0)))
    at, mvt = _tc_call(qiT_p, mk_p, mv_p)
    out = _get_sc_call()(at, mvt)
    return jnp.swapaxes(out, 0, 1).reshape(1, CV, 30, 30)


# speedup vs baseline: 35.8350x; 1.0137x over previous
import functools
import math

import jax
import jax.numpy as jnp
from jax import lax
from jax.experimental import pallas as pl
from jax.experimental.pallas import tpu as pltpu
from jax.experimental.pallas import tpu_sc as plsc

CK = 64
CV = 128
HW = 900
M = 36000
TOPK = 50

NPAD = 36096
G = 141
H2 = 9
MROWS = 904
NEG = -1e30

NC, NS, LANES = 2, 16, 16
NW = NC * NS
COLS_PER_W = 29

NTILE = 256


def _tc_body(q_ref, k_ref, v_ref, at_ref, mvt_ref, gm_ref, gms):
    at = lax.dot_general(
        q_ref[...], k_ref[...], (((1,), (0,)), ((), ())),
        preferred_element_type=jnp.float32)
    at_ref[...] = at
    mvt_ref[...] = v_ref[...].T
    i = pl.program_id(0)
    lim = jnp.minimum(NTILE, M - i * NTILE)
    lanes = lax.broadcasted_iota(jnp.int32, (MROWS, NTILE), 1)
    gmcol = jnp.max(jnp.where(lanes < lim, at, NEG), axis=1, keepdims=True)
    slot = lax.broadcasted_iota(jnp.int32, (MROWS, H2 * LANES), 1)
    acc = jnp.where(slot == i, jnp.broadcast_to(gmcol, (MROWS, H2 * LANES)),
                    jnp.full((MROWS, H2 * LANES), NEG, jnp.float32))

    @pl.when(i == 0)
    def _():
        gms[...] = acc

    @pl.when(i > 0)
    def _():
        gms[...] = jnp.maximum(gms[...], acc)

    @pl.when(i == NPAD // NTILE - 1)
    def _():
        gm_ref[...] = gms[...]


_tc_call = pl.pallas_call(
    _tc_body,
    grid=(NPAD // NTILE,),
    in_specs=[
        pl.BlockSpec((MROWS, CK), lambda i: (0, 0)),
        pl.BlockSpec((CK, NTILE), lambda i: (0, i)),
        pl.BlockSpec((CV, NTILE), lambda i: (0, i)),
    ],
    out_specs=[
        pl.BlockSpec((MROWS, NTILE), lambda i: (0, i)),
        pl.BlockSpec((NTILE, CV), lambda i: (i, 0)),
        pl.BlockSpec((MROWS, H2 * LANES), lambda i: (0, 0)),
    ],
    out_shape=[
        jax.ShapeDtypeStruct((MROWS, NPAD), jnp.float32),
        jax.ShapeDtypeStruct((NPAD, CV), jnp.float32),
        jax.ShapeDtypeStruct((MROWS, H2 * LANES), jnp.float32),
    ],
    scratch_shapes=[pltpu.VMEM((MROWS, H2 * LANES), jnp.float32)],
)


def _lane(vec, lane_scalar):
    idx = lax.broadcast(lane_scalar, (LANES,))
    return jnp.take_along_axis(vec, idx, axis=0, mode="promise_in_bounds")[0]


def _sc_body(at_hbm, mvt_hbm, gm_hbm, out_hbm, colbuf, gmb, tv, ti, rows,
             orow, sem):
    wid = lax.axis_index("s") * NC + lax.axis_index("c")
    iota = lax.iota(jnp.int32, LANES)
    lane0 = iota == 0
    neg = jnp.full((LANES,), NEG, jnp.float32)

    def do_col(col):
        pltpu.sync_copy(at_hbm.at[pl.ds(col, 1)], colbuf)
        pltpu.sync_copy(gm_hbm.at[pl.ds(col, 1)], gmb)
        for j in range(M // LANES, NPAD // LANES):
            colbuf[0, pl.ds(j * LANES, LANES)] = neg
        g8 = gmb[0, pl.ds(8 * LANES, LANES)]
        gmb[0, pl.ds(8 * LANES, LANES)] = jnp.where(iota >= G - 128, NEG, g8)

        for c in range(4):
            tv[pl.ds(c * LANES, LANES)] = neg
            ti[pl.ds(c * LANES, LANES)] = jnp.zeros((LANES,), jnp.int32)

        def extract(t, _):
            glob = gmb[0, pl.ds(0, LANES)]
            for h in range(1, H2):
                glob = jnp.maximum(glob, gmb[0, pl.ds(h * LANES, LANES)])
            m = jnp.max(glob, axis=0)
            mvec = lax.broadcast(m, (LANES,))

            genc = jnp.full((LANES,), 9999, jnp.int32)
            for h in range(H2):
                hit = gmb[0, pl.ds(h * LANES, LANES)] == mvec
                genc = jnp.where(hit, jnp.minimum(genc, iota + h * LANES), genc)
            gstar = jnp.min(genc, axis=0)
            base = gstar * 256

            kmin = jnp.full((LANES,), 999, jnp.int32)
            for k in range(16):
                hit = colbuf[0, pl.ds(base + k * LANES, LANES)] == mvec
                kmin = jnp.where(hit, jnp.minimum(kmin, k), kmin)
            off = jnp.min(kmin * LANES + iota, axis=0)
            kstar = lax.shift_right_logical(off, 4)
            lstar = lax.bitwise_and(off, 15)

            tvec = lax.broadcast(t, (LANES,))
            plsc.store_scatter(tv, [tvec], mvec, mask=lane0)
            plsc.store_scatter(
                ti, [tvec], lax.broadcast(base + off, (LANES,)), mask=lane0)

            v = colbuf[0, pl.ds(base + kstar * LANES, LANES)]
            colbuf[0, pl.ds(base + kstar * LANES, LANES)] = (
                jnp.where(iota == lax.broadcast(lstar, (LANES,)), NEG, v))
            mx = colbuf[0, pl.ds(base, LANES)]
            for k in range(1, 16):
                mx = jnp.maximum(mx, colbuf[0, pl.ds(base + k * LANES, LANES)])
            plsc.store_scatter(
                gmb, [jnp.zeros((LANES,), jnp.int32),
                      lax.broadcast(gstar, (LANES,))],
                lax.broadcast(jnp.max(mx, axis=0), (LANES,)), mask=lane0)
            return 0

        lax.fori_loop(0, TOPK, extract, 0)

        v0 = lax.broadcast(tv[pl.ds(0, LANES)][0], (LANES,))
        wvecs = []
        zacc = jnp.zeros((LANES,), jnp.float32)
        for c in range(4):
            e = jnp.exp(tv[pl.ds(c * LANES, LANES)] - v0)
            wvecs.append(e)
            zacc = zacc + e
        zvec = lax.broadcast(jnp.sum(zacc, axis=0), (LANES,))
        wvecs = [w / zvec for w in wvecs]

        pltpu.async_copy(mvt_hbm.at[ti], rows, sem).wait()

        accs = [jnp.zeros((LANES,), jnp.float32) for _ in range(CV // LANES)]
        for k in range(TOPK):
            w = lax.broadcast(wvecs[k // LANES][k % LANES], (LANES,))
            for c in range(CV // LANES):
                accs[c] = accs[c] + w * rows[k, pl.ds(c * LANES, LANES)]
        for c in range(CV // LANES):
            orow[0, pl.ds(c * LANES, LANES)] = accs[c]
        pltpu.sync_copy(orow, out_hbm.at[pl.ds(col, 1)])

    def col_body(i, _):
        col = i * NW + wid

        @pl.when(col < HW)
        def _():
            do_col(col)

        return 0

    lax.fori_loop(0, COLS_PER_W, col_body, 0)


@functools.cache
def _get_sc_call():
    return pl.kernel(
        _sc_body,
        out_type=jax.ShapeDtypeStruct((HW, CV), jnp.float32),
        mesh=plsc.VectorSubcoreMesh(
            core_axis_name="c", subcore_axis_name="s",
            num_cores=NC, num_subcores=NS),
        compiler_params=pltpu.CompilerParams(needs_layout_passes=False),
        scratch_types=[
            pltpu.VMEM((1, NPAD), jnp.float32),
            pltpu.VMEM((1, H2 * LANES), jnp.float32),
            pltpu.VMEM((64,), jnp.float32),
            pltpu.VMEM((64,), jnp.int32),
            pltpu.VMEM((64, CV), jnp.float32),
            pltpu.VMEM((1, CV), jnp.float32),
            pltpu.SemaphoreType.DMA,
        ],
    )


def kernel(mk, mv, qk):
    mk_f = mk.reshape(CK, M)
    mv_f = mv.reshape(CV, M)
    qiT = jnp.swapaxes(qk.reshape(CK, HW), 0, 1) / math.sqrt(CK)
    mk_p = jnp.pad(mk_f, ((0, 0), (0, NPAD - M)))
    mv_p = jnp.pad(mv_f, ((0, 0), (0, NPAD - M)))
    qiT_p = jnp.pad(qiT, ((0, MROWS - HW), (0, 0)))
    at, mvt, gm = _tc_call(qiT_p, mk_p, mv_p)
    out = _get_sc_call()(at, mvt, gm)
    return jnp.swapaxes(out, 0, 1).reshape(1, CV, 30, 30)


# speedup vs baseline: 41.4165x; 1.1558x over previous
import functools
import math

import jax
import jax.numpy as jnp
from jax import lax
from jax.experimental import pallas as pl
from jax.experimental.pallas import tpu as pltpu
from jax.experimental.pallas import tpu_sc as plsc

CK = 64
CV = 128
HW = 900
M = 36000
TOPK = 50

NPAD = 36096
G = 141
H2 = 9
MROWS = 904
NEG = -1e30

NC, NS, LANES = 2, 16, 16
NW = NC * NS
COLS_PER_W = 29

NTILE = 256


def _tc_body(q_ref, k_ref, v_ref, at_ref, mvt_ref, gm_ref, gms):
    at = lax.dot_general(
        q_ref[...], k_ref[...], (((1,), (0,)), ((), ())),
        preferred_element_type=jnp.float32)
    i = pl.program_id(0)
    lim = jnp.minimum(NTILE, M - i * NTILE)
    lanes = lax.broadcasted_iota(jnp.int32, (MROWS, NTILE), 1)
    at = jnp.where(lanes < lim, at, NEG)
    at_ref[...] = at
    mvt_ref[...] = v_ref[...].T
    gmcol = jnp.max(at, axis=1, keepdims=True)
    slot = lax.broadcasted_iota(jnp.int32, (MROWS, H2 * LANES), 1)
    acc = jnp.where(slot == i, jnp.broadcast_to(gmcol, (MROWS, H2 * LANES)),
                    jnp.full((MROWS, H2 * LANES), NEG, jnp.float32))

    @pl.when(i == 0)
    def _():
        gms[...] = acc

    @pl.when(i > 0)
    def _():
        gms[...] = jnp.maximum(gms[...], acc)

    @pl.when(i == NPAD // NTILE - 1)
    def _():
        gm_ref[...] = gms[...]


_tc_call = pl.pallas_call(
    _tc_body,
    grid=(NPAD // NTILE,),
    in_specs=[
        pl.BlockSpec((MROWS, CK), lambda i: (0, 0)),
        pl.BlockSpec((CK, NTILE), lambda i: (0, i)),
        pl.BlockSpec((CV, NTILE), lambda i: (0, i)),
    ],
    out_specs=[
        pl.BlockSpec((MROWS, NTILE), lambda i: (0, i)),
        pl.BlockSpec((NTILE, CV), lambda i: (i, 0)),
        pl.BlockSpec((MROWS, H2 * LANES), lambda i: (0, 0)),
    ],
    out_shape=[
        jax.ShapeDtypeStruct((MROWS, NPAD), jnp.float32),
        jax.ShapeDtypeStruct((NPAD, CV), jnp.float32),
        jax.ShapeDtypeStruct((MROWS, H2 * LANES), jnp.float32),
    ],
    scratch_shapes=[pltpu.VMEM((MROWS, H2 * LANES), jnp.float32)],
)


def _sc_body(at2_hbm, mvt_hbm, gm_hbm, out_hbm, gmb, gidb, smx, grp, tv, ti,
             rows, orow, sem, gsem):
    wid = lax.axis_index("s") * NC + lax.axis_index("c")
    iota = lax.iota(jnp.int32, LANES)
    lane0 = iota == 0
    neg = jnp.full((LANES,), NEG, jnp.float32)

    def do_col(col, valid):
        pltpu.sync_copy(gm_hbm.at[pl.ds(col, 1)], gmb)
        cbase = col * G

        for c in range(4):
            smx[pl.ds(c * LANES, LANES)] = neg
            tv[pl.ds(c * LANES, LANES)] = neg
            ti[pl.ds(c * LANES, LANES)] = jnp.zeros((LANES,), jnp.int32)
            gidb[pl.ds(c * LANES, LANES)] = lax.broadcast(cbase, (LANES,))

        def pick_groups(t, gs):
            glob = gs[0]
            for h in range(1, H2):
                glob = jnp.maximum(glob, gs[h])
            m = jnp.max(glob, axis=0)
            mvec = lax.broadcast(m, (LANES,))
            genc = jnp.full((LANES,), 9999, jnp.int32)
            for h in range(H2):
                genc = jnp.where(gs[h] == mvec,
                                 jnp.minimum(genc, iota + h * LANES), genc)
            gstar = jnp.min(genc, axis=0)
            tvec = lax.broadcast(t, (LANES,))
            plsc.store_scatter(smx, [tvec], mvec, mask=lane0)
            plsc.store_scatter(
                gidb, [tvec], lax.broadcast(cbase + gstar, (LANES,)),
                mask=lane0)
            gsv = lax.broadcast(gstar, (LANES,))
            return tuple(
                jnp.where(iota + h * LANES == gsv, NEG, gs[h])
                for h in range(H2))

        lax.fori_loop(
            0, TOPK, pick_groups,
            tuple(gmb[0, pl.ds(h * LANES, LANES)] for h in range(H2)))

        pltpu.async_copy(at2_hbm.at[gidb], grp, gsem).wait()

        def extract(t, ss):
            glob = jnp.maximum(jnp.maximum(ss[0], ss[1]),
                               jnp.maximum(ss[2], ss[3]))
            m = jnp.max(glob, axis=0)
            mvec = lax.broadcast(m, (LANES,))
            senc = jnp.full((LANES,), 999, jnp.int32)
            for c in range(4):
                senc = jnp.where(ss[c] == mvec,
                                 jnp.minimum(senc, iota + c * LANES), senc)
            sstar = jnp.min(senc, axis=0)
            base = sstar * 256

            kmin = jnp.full((LANES,), 999, jnp.int32)
            for k in range(16):
                hit = grp[sstar, pl.ds(k * LANES, LANES)] == mvec
                kmin = jnp.where(hit, jnp.minimum(kmin, k), kmin)
            off = jnp.min(kmin * LANES + iota, axis=0)
            kstar = lax.shift_right_logical(off, 4)
            lstar = lax.bitwise_and(off, 15)

            tvec = lax.broadcast(t, (LANES,))
            plsc.store_scatter(tv, [tvec], mvec, mask=lane0)
            plsc.store_scatter(
                ti, [tvec], lax.broadcast(base + off, (LANES,)), mask=lane0)

            v = grp[sstar, pl.ds(kstar * LANES, LANES)]
            grp[sstar, pl.ds(kstar * LANES, LANES)] = (
                jnp.where(iota == lax.broadcast(lstar, (LANES,)), NEG, v))
            mx = grp[sstar, pl.ds(0, LANES)]
            for k in range(1, 16):
                mx = jnp.maximum(mx, grp[sstar, pl.ds(k * LANES, LANES)])
            nmx = lax.broadcast(jnp.max(mx, axis=0), (LANES,))
            ssv = lax.broadcast(sstar, (LANES,))
            return tuple(
                jnp.where(iota + c * LANES == ssv, nmx, ss[c])
                for c in range(4))

        lax.fori_loop(
            0, TOPK, extract,
            tuple(smx[pl.ds(c * LANES, LANES)] for c in range(4)))

        cbv = lax.broadcast(cbase, (LANES,))
        for c in range(4):
            sv = ti[pl.ds(c * LANES, LANES)]
            slot = lax.shift_right_logical(sv, 8)
            off = lax.bitwise_and(sv, 255)
            gg = plsc.load_gather(gidb, [slot])
            ti[pl.ds(c * LANES, LANES)] = (gg - cbv) * 256 + off

        v0 = lax.broadcast(tv[pl.ds(0, LANES)][0], (LANES,))
        wvecs = []
        zacc = jnp.zeros((LANES,), jnp.float32)
        for c in range(4):
            e = jnp.exp(tv[pl.ds(c * LANES, LANES)] - v0)
            wvecs.append(e)
            zacc = zacc + e
        zvec = lax.broadcast(jnp.sum(zacc, axis=0), (LANES,))
        wvecs = [w / zvec for w in wvecs]

        pltpu.async_copy(mvt_hbm.at[ti], rows, sem).wait()

        accs = [jnp.zeros((LANES,), jnp.float32) for _ in range(CV // LANES)]
        for k in range(TOPK):
            w = lax.broadcast(wvecs[k // LANES][k % LANES], (LANES,))
            for c in range(CV // LANES):
                accs[c] = accs[c] + w * rows[k, pl.ds(c * LANES, LANES)]
        for c in range(CV // LANES):
            orow[0, pl.ds(c * LANES, LANES)] = accs[c]

        @pl.when(valid)
        def _():
            pltpu.sync_copy(orow, out_hbm.at[pl.ds(col, 1)])

    def col_body(i, _):
        col = i * NW + wid
        do_col(jnp.minimum(col, HW - 1), col < HW)
        return 0

    lax.fori_loop(0, COLS_PER_W, col_body, 0)


@functools.cache
def _get_sc_call():
    return pl.kernel(
        _sc_body,
        out_type=jax.ShapeDtypeStruct((HW, CV), jnp.float32),
        mesh=plsc.VectorSubcoreMesh(
            core_axis_name="c", subcore_axis_name="s",
            num_cores=NC, num_subcores=NS),
        compiler_params=pltpu.CompilerParams(needs_layout_passes=False),
        scratch_types=[
            pltpu.VMEM((1, H2 * LANES), jnp.float32),
            pltpu.VMEM((64,), jnp.int32),
            pltpu.VMEM((64,), jnp.float32),
            pltpu.VMEM((64, NTILE), jnp.float32),
            pltpu.VMEM((64,), jnp.float32),
            pltpu.VMEM((64,), jnp.int32),
            pltpu.VMEM((64, CV), jnp.float32),
            pltpu.VMEM((1, CV), jnp.float32),
            pltpu.SemaphoreType.DMA,
            pltpu.SemaphoreType.DMA,
        ],
    )


def kernel(mk, mv, qk):
    mk_f = mk.reshape(CK, M)
    mv_f = mv.reshape(CV, M)
    qiT = jnp.swapaxes(qk.reshape(CK, HW), 0, 1) / math.sqrt(CK)
    mk_p = jnp.pad(mk_f, ((0, 0), (0, NPAD - M)))
    mv_p = jnp.pad(mv_f, ((0, 0), (0, NPAD - M)))
    qiT_p = jnp.pad(qiT, ((0, MROWS - HW), (0, 0)))
    at, mvt, gm = _tc_call(qiT_p, mk_p, mv_p)
    at2 = at.reshape(MROWS * G, NTILE)
    out = _get_sc_call()(at2, mvt, gm)
    return jnp.swapaxes(out, 0, 1).reshape(1, CV, 30, 30)


# speedup vs baseline: 50.9387x; 1.2299x over previous
import functools
import math

import jax
import jax.numpy as jnp
from jax import lax
from jax.experimental import pallas as pl
from jax.experimental.pallas import tpu as pltpu
from jax.experimental.pallas import tpu_sc as plsc

CK = 64
CV = 128
HW = 900
M = 36000
TOPK = 50

NPAD = 36096
G = 141
H2 = 9
MROWS = 904
NEG = -1e30

NC, NS, LANES = 2, 16, 16
NW = NC * NS
COLS_PER_W = 29

NTILE = 256


def _tc_body(q_ref, k_ref, v_ref, at_ref, mvt_ref, gm_ref, gms):
    at = lax.dot_general(
        q_ref[...], k_ref[...], (((1,), (0,)), ((), ())),
        preferred_element_type=jnp.float32)
    i = pl.program_id(0)
    lim = jnp.minimum(NTILE, M - i * NTILE)
    lanes = lax.broadcasted_iota(jnp.int32, (MROWS, NTILE), 1)
    at = jnp.where(lanes < lim, at, NEG)
    at_ref[...] = at
    mvt_ref[...] = v_ref[...].T
    gmcol = jnp.max(at, axis=1, keepdims=True)
    slot = lax.broadcasted_iota(jnp.int32, (MROWS, H2 * LANES), 1)
    acc = jnp.where(slot == i, jnp.broadcast_to(gmcol, (MROWS, H2 * LANES)),
                    jnp.full((MROWS, H2 * LANES), NEG, jnp.float32))

    @pl.when(i == 0)
    def _():
        gms[...] = acc

    @pl.when(i > 0)
    def _():
        gms[...] = jnp.maximum(gms[...], acc)

    @pl.when(i == NPAD // NTILE - 1)
    def _():
        gm_ref[...] = gms[...]


_tc_call = pl.pallas_call(
    _tc_body,
    grid=(NPAD // NTILE,),
    in_specs=[
        pl.BlockSpec((MROWS, CK), lambda i: (0, 0)),
        pl.BlockSpec((CK, NTILE), lambda i: (0, i)),
        pl.BlockSpec((CV, NTILE), lambda i: (0, i)),
    ],
    out_specs=[
        pl.BlockSpec((MROWS, NTILE), lambda i: (i, 0)),
        pl.BlockSpec((NTILE, CV), lambda i: (i, 0)),
        pl.BlockSpec((MROWS, H2 * LANES), lambda i: (0, 0)),
    ],
    out_shape=[
        jax.ShapeDtypeStruct((G * MROWS, NTILE), jnp.float32),
        jax.ShapeDtypeStruct((NPAD, CV), jnp.float32),
        jax.ShapeDtypeStruct((MROWS, H2 * LANES), jnp.float32),
    ],
    scratch_shapes=[pltpu.VMEM((MROWS, H2 * LANES), jnp.float32)],
)


def _sc_body(at2_hbm, mvt_hbm, gm_hbm, out_hbm, gmb, gidb, gidl, smx, grp,
             tv, ti, rows, orow, sem, gsem):
    wid = lax.axis_index("s") * NC + lax.axis_index("c")
    iota = lax.iota(jnp.int32, LANES)
    lane0 = iota == 0
    neg = jnp.full((LANES,), NEG, jnp.float32)

    def do_col(col, valid):
        pltpu.sync_copy(gm_hbm.at[pl.ds(col, 1)], gmb)

        for c in range(4):
            smx[pl.ds(c * LANES, LANES)] = neg
            tv[pl.ds(c * LANES, LANES)] = neg
            ti[pl.ds(c * LANES, LANES)] = jnp.zeros((LANES,), jnp.int32)
            gidb[pl.ds(c * LANES, LANES)] = lax.broadcast(col, (LANES,))
            gidl[pl.ds(c * LANES, LANES)] = jnp.zeros((LANES,), jnp.int32)

        def pick_groups(t, gs):
            glob = gs[0]
            for h in range(1, H2):
                glob = jnp.maximum(glob, gs[h])
            m = jnp.max(glob, axis=0)
            mvec = lax.broadcast(m, (LANES,))
            genc = jnp.full((LANES,), 9999, jnp.int32)
            for h in range(H2):
                genc = jnp.where(gs[h] == mvec,
                                 jnp.minimum(genc, iota + h * LANES), genc)
            gstar = jnp.min(genc, axis=0)
            tvec = lax.broadcast(t, (LANES,))
            plsc.store_scatter(smx, [tvec], mvec, mask=lane0)
            plsc.store_scatter(
                gidb, [tvec], lax.broadcast(gstar * MROWS + col, (LANES,)),
                mask=lane0)
            plsc.store_scatter(
                gidl, [tvec], lax.broadcast(gstar, (LANES,)), mask=lane0)
            gsv = lax.broadcast(gstar, (LANES,))
            return tuple(
                jnp.where(iota + h * LANES == gsv, NEG, gs[h])
                for h in range(H2))

        lax.fori_loop(
            0, TOPK, pick_groups,
            tuple(gmb[0, pl.ds(h * LANES, LANES)] for h in range(H2)))

        pltpu.async_copy(at2_hbm.at[gidb], grp, gsem).wait()

        def extract(t, ss):
            glob = jnp.maximum(jnp.maximum(ss[0], ss[1]),
                               jnp.maximum(ss[2], ss[3]))
            m = jnp.max(glob, axis=0)
            mvec = lax.broadcast(m, (LANES,))
            senc = jnp.full((LANES,), 999, jnp.int32)
            for c in range(4):
                senc = jnp.where(ss[c] == mvec,
                                 jnp.minimum(senc, iota + c * LANES), senc)
            sstar = jnp.min(senc, axis=0)
            base = sstar * 256

            kmin = jnp.full((LANES,), 999, jnp.int32)
            for k in range(16):
                hit = grp[sstar, pl.ds(k * LANES, LANES)] == mvec
                kmin = jnp.where(hit, jnp.minimum(kmin, k), kmin)
            off = jnp.min(kmin * LANES + iota, axis=0)
            kstar = lax.shift_right_logical(off, 4)
            lstar = lax.bitwise_and(off, 15)

            tvec = lax.broadcast(t, (LANES,))
            plsc.store_scatter(tv, [tvec], mvec, mask=lane0)
            plsc.store_scatter(
                ti, [tvec], lax.broadcast(base + off, (LANES,)), mask=lane0)

            v = grp[sstar, pl.ds(kstar * LANES, LANES)]
            grp[sstar, pl.ds(kstar * LANES, LANES)] = (
                jnp.where(iota == lax.broadcast(lstar, (LANES,)), NEG, v))
            mx = grp[sstar, pl.ds(0, LANES)]
            for k in range(1, 16):
                mx = jnp.maximum(mx, grp[sstar, pl.ds(k * LANES, LANES)])
            nmx = lax.broadcast(jnp.max(mx, axis=0), (LANES,))
            ssv = lax.broadcast(sstar, (LANES,))
            return tuple(
                jnp.where(iota + c * LANES == ssv, nmx, ss[c])
                for c in range(4))

        lax.fori_loop(
            0, TOPK, extract,
            tuple(smx[pl.ds(c * LANES, LANES)] for c in range(4)))

        for c in range(4):
            sv = ti[pl.ds(c * LANES, LANES)]
            slot = lax.shift_right_logical(sv, 8)
            off = lax.bitwise_and(sv, 255)
            gg = plsc.load_gather(gidl, [slot])
            ti[pl.ds(c * LANES, LANES)] = gg * 256 + off

        v0 = lax.broadcast(tv[pl.ds(0, LANES)][0], (LANES,))
        wvecs = []
        zacc = jnp.zeros((LANES,), jnp.float32)
        for c in range(4):
            e = jnp.exp(tv[pl.ds(c * LANES, LANES)] - v0)
            wvecs.append(e)
            zacc = zacc + e
        zvec = lax.broadcast(jnp.sum(zacc, axis=0), (LANES,))
        wvecs = [w / zvec for w in wvecs]

        pltpu.async_copy(mvt_hbm.at[ti], rows, sem).wait()

        accs = [jnp.zeros((LANES,), jnp.float32) for _ in range(CV // LANES)]
        for k in range(TOPK):
            w = lax.broadcast(wvecs[k // LANES][k % LANES], (LANES,))
            for c in range(CV // LANES):
                accs[c] = accs[c] + w * rows[k, pl.ds(c * LANES, LANES)]
        for c in range(CV // LANES):
            orow[0, pl.ds(c * LANES, LANES)] = accs[c]

        @pl.when(valid)
        def _():
            pltpu.sync_copy(orow, out_hbm.at[pl.ds(col, 1)])

    def col_body(i, _):
        col = i * NW + wid
        do_col(jnp.minimum(col, HW - 1), col < HW)
        return 0

    lax.fori_loop(0, COLS_PER_W, col_body, 0)


@functools.cache
def _get_sc_call():
    return pl.kernel(
        _sc_body,
        out_type=jax.ShapeDtypeStruct((HW, CV), jnp.float32),
        mesh=plsc.VectorSubcoreMesh(
            core_axis_name="c", subcore_axis_name="s",
            num_cores=NC, num_subcores=NS),
        compiler_params=pltpu.CompilerParams(needs_layout_passes=False),
        scratch_types=[
            pltpu.VMEM((1, H2 * LANES), jnp.float32),
            pltpu.VMEM((64,), jnp.int32),
            pltpu.VMEM((64,), jnp.int32),
            pltpu.VMEM((64,), jnp.float32),
            pltpu.VMEM((64, NTILE), jnp.float32),
            pltpu.VMEM((64,), jnp.float32),
            pltpu.VMEM((64,), jnp.int32),
            pltpu.VMEM((64, CV), jnp.float32),
            pltpu.VMEM((1, CV), jnp.float32),
            pltpu.SemaphoreType.DMA,
            pltpu.SemaphoreType.DMA,
        ],
    )


def kernel(mk, mv, qk):
    mk_f = mk.reshape(CK, M)
    mv_f = mv.reshape(CV, M)
    qiT = jnp.swapaxes(qk.reshape(CK, HW), 0, 1) / math.sqrt(CK)
    qiT_p = jnp.pad(qiT, ((0, MROWS - HW), (0, 0)))
    at2, mvt, gm = _tc_call(qiT_p, mk_f, mv_f)
    out = _get_sc_call()(at2, mvt, gm)
    return jnp.swapaxes(out, 0, 1).reshape(1, CV, 30, 30)


# speedup vs baseline: 51.1050x; 1.0033x over previous
import functools
import math

import jax
import jax.numpy as jnp
from jax import lax
from jax.experimental import pallas as pl
from jax.experimental.pallas import tpu as pltpu
from jax.experimental.pallas import tpu_sc as plsc

CK = 64
CV = 128
HW = 900
M = 36000
TOPK = 50

NPAD = 36096
G = 141
H2 = 9
GMW = 256
MROWS = 904
NEG = -1e30

NC, NS, LANES = 2, 16, 16
NW = NC * NS
COLS_PER_W = 29

NTILE = 256


def _tc_body(q_ref, k_ref, v_ref, at_ref, mvt_ref, gm_ref, gms):
    at = lax.dot_general(
        q_ref[...], k_ref[...], (((1,), (0,)), ((), ())),
        preferred_element_type=jnp.float32)
    i = pl.program_id(0)
    lim = jnp.minimum(NTILE, M - i * NTILE)
    lanes = lax.broadcasted_iota(jnp.int32, (MROWS, NTILE), 1)
    at = jnp.where(lanes < lim, at, NEG)
    at_ref[...] = at
    mvt_ref[...] = v_ref[...].T
    gmcol = jnp.max(at, axis=1, keepdims=True)
    slot = lax.broadcasted_iota(jnp.int32, (MROWS, GMW), 1)
    acc = jnp.where(slot == i, jnp.broadcast_to(gmcol, (MROWS, GMW)),
                    jnp.full((MROWS, GMW), NEG, jnp.float32))

    @pl.when(i == 0)
    def _():
        gms[...] = acc

    @pl.when(i > 0)
    def _():
        gms[...] = jnp.maximum(gms[...], acc)

    @pl.when(i == NPAD // NTILE - 1)
    def _():
        gm_ref[...] = gms[...]


_tc_call = pl.pallas_call(
    _tc_body,
    grid=(NPAD // NTILE,),
    in_specs=[
        pl.BlockSpec((MROWS, CK), lambda i: (0, 0)),
        pl.BlockSpec((CK, NTILE), lambda i: (0, i)),
        pl.BlockSpec((CV, NTILE), lambda i: (0, i)),
    ],
    out_specs=[
        pl.BlockSpec((MROWS, NTILE), lambda i: (i, 0)),
        pl.BlockSpec((NTILE, CV), lambda i: (i, 0)),
        pl.BlockSpec((MROWS, GMW), lambda i: (0, 0)),
    ],
    out_shape=[
        jax.ShapeDtypeStruct((G * MROWS, NTILE), jnp.float32),
        jax.ShapeDtypeStruct((NPAD, CV), jnp.float32),
        jax.ShapeDtypeStruct((MROWS, GMW), jnp.float32),
    ],
    scratch_shapes=[pltpu.VMEM((MROWS, GMW), jnp.float32)],
)


def _sc_body(at2_hbm, mvt_hbm, gm_hbm, out_hbm, cidb, gmball, gidb, gidl,
             smx, grp, tv, ti, rows, orow, sem, gsem):
    wid = lax.axis_index("s") * NC + lax.axis_index("c")
    iota = lax.iota(jnp.int32, LANES)
    lane0 = iota == 0
    neg = jnp.full((LANES,), NEG, jnp.float32)

    for c in range(2):
        cidb[pl.ds(c * LANES, LANES)] = jnp.minimum(
            (iota + c * LANES) * NW + wid, HW - 1)
    pltpu.async_copy(gm_hbm.at[cidb], gmball, sem).wait()

    def do_col(i, col, valid):

        for c in range(4):
            smx[pl.ds(c * LANES, LANES)] = neg
            tv[pl.ds(c * LANES, LANES)] = neg
            ti[pl.ds(c * LANES, LANES)] = jnp.zeros((LANES,), jnp.int32)
            gidb[pl.ds(c * LANES, LANES)] = lax.broadcast(col, (LANES,))
            gidl[pl.ds(c * LANES, LANES)] = jnp.zeros((LANES,), jnp.int32)

        def pick_groups(t, gs):
            glob = gs[0]
            for h in range(1, H2):
                glob = jnp.maximum(glob, gs[h])
            m = jnp.max(glob, axis=0)
            mvec = lax.broadcast(m, (LANES,))
            genc = jnp.full((LANES,), 9999, jnp.int32)
            for h in range(H2):
                genc = jnp.where(gs[h] == mvec,
                                 jnp.minimum(genc, iota + h * LANES), genc)
            gstar = jnp.min(genc, axis=0)
            tvec = lax.broadcast(t, (LANES,))
            plsc.store_scatter(smx, [tvec], mvec, mask=lane0)
            plsc.store_scatter(
                gidb, [tvec], lax.broadcast(gstar * MROWS + col, (LANES,)),
                mask=lane0)
            plsc.store_scatter(
                gidl, [tvec], lax.broadcast(gstar, (LANES,)), mask=lane0)
            gsv = lax.broadcast(gstar, (LANES,))
            return tuple(
                jnp.where(iota + h * LANES == gsv, NEG, gs[h])
                for h in range(H2))

        lax.fori_loop(
            0, TOPK, pick_groups,
            tuple(gmball[i, pl.ds(h * LANES, LANES)] for h in range(H2)))

        pltpu.async_copy(at2_hbm.at[gidb], grp, gsem).wait()

        def extract(t, ss):
            glob = jnp.maximum(jnp.maximum(ss[0], ss[1]),
                               jnp.maximum(ss[2], ss[3]))
            m = jnp.max(glob, axis=0)
            mvec = lax.broadcast(m, (LANES,))
            senc = jnp.full((LANES,), 999, jnp.int32)
            for c in range(4):
                senc = jnp.where(ss[c] == mvec,
                                 jnp.minimum(senc, iota + c * LANES), senc)
            sstar = jnp.min(senc, axis=0)
            base = sstar * 256

            kmin = jnp.full((LANES,), 999, jnp.int32)
            for k in range(16):
                hit = grp[sstar, pl.ds(k * LANES, LANES)] == mvec
                kmin = jnp.where(hit, jnp.minimum(kmin, k), kmin)
            off = jnp.min(kmin * LANES + iota, axis=0)
            kstar = lax.shift_right_logical(off, 4)
            lstar = lax.bitwise_and(off, 15)

            tvec = lax.broadcast(t, (LANES,))
            plsc.store_scatter(tv, [tvec], mvec, mask=lane0)
            plsc.store_scatter(
                ti, [tvec], lax.broadcast(base + off, (LANES,)), mask=lane0)

            v = grp[sstar, pl.ds(kstar * LANES, LANES)]
            grp[sstar, pl.ds(kstar * LANES, LANES)] = (
                jnp.where(iota == lax.broadcast(lstar, (LANES,)), NEG, v))
            mx = grp[sstar, pl.ds(0, LANES)]
            for k in range(1, 16):
                mx = jnp.maximum(mx, grp[sstar, pl.ds(k * LANES, LANES)])
            nmx = lax.broadcast(jnp.max(mx, axis=0), (LANES,))
            ssv = lax.broadcast(sstar, (LANES,))
            return tuple(
                jnp.where(iota + c * LANES == ssv, nmx, ss[c])
                for c in range(4))

        lax.fori_loop(
            0, TOPK, extract,
            tuple(smx[pl.ds(c * LANES, LANES)] for c in range(4)))

        for c in range(4):
            sv = ti[pl.ds(c * LANES, LANES)]
            slot = lax.shift_right_logical(sv, 8)
            off = lax.bitwise_and(sv, 255)
            gg = plsc.load_gather(gidl, [slot])
            ti[pl.ds(c * LANES, LANES)] = gg * 256 + off

        v0 = lax.broadcast(tv[pl.ds(0, LANES)][0], (LANES,))
        wvecs = []
        zacc = jnp.zeros((LANES,), jnp.float32)
        for c in range(4):
            e = jnp.exp(tv[pl.ds(c * LANES, LANES)] - v0)
            wvecs.append(e)
            zacc = zacc + e
        zvec = lax.broadcast(jnp.sum(zacc, axis=0), (LANES,))
        wvecs = [w / zvec for w in wvecs]

        pltpu.async_copy(mvt_hbm.at[ti], rows, sem).wait()

        accs = [jnp.zeros((LANES,), jnp.float32) for _ in range(CV // LANES)]
        for k in range(TOPK):
            w = lax.broadcast(wvecs[k // LANES][k % LANES], (LANES,))
            for c in range(CV // LANES):
                accs[c] = accs[c] + w * rows[k, pl.ds(c * LANES, LANES)]
        for c in range(CV // LANES):
            orow[0, pl.ds(c * LANES, LANES)] = accs[c]

        @pl.when(valid)
        def _():
            pltpu.sync_copy(orow, out_hbm.at[pl.ds(col, 1)])

    def col_body(i, _):
        col = i * NW + wid
        do_col(i, jnp.minimum(col, HW - 1), col < HW)
        return 0

    lax.fori_loop(0, COLS_PER_W, col_body, 0)


@functools.cache
def _get_sc_call():
    return pl.kernel(
        _sc_body,
        out_type=jax.ShapeDtypeStruct((HW, CV), jnp.float32),
        mesh=plsc.VectorSubcoreMesh(
            core_axis_name="c", subcore_axis_name="s",
            num_cores=NC, num_subcores=NS),
        compiler_params=pltpu.CompilerParams(needs_layout_passes=False),
        scratch_types=[
            pltpu.VMEM((2 * LANES,), jnp.int32),
            pltpu.VMEM((2 * LANES, GMW), jnp.float32),
            pltpu.VMEM((64,), jnp.int32),
            pltpu.VMEM((64,), jnp.int32),
            pltpu.VMEM((64,), jnp.float32),
            pltpu.VMEM((64, NTILE), jnp.float32),
            pltpu.VMEM((64,), jnp.float32),
            pltpu.VMEM((64,), jnp.int32),
            pltpu.VMEM((64, CV), jnp.float32),
            pltpu.VMEM((1, CV), jnp.float32),
            pltpu.SemaphoreType.DMA,
            pltpu.SemaphoreType.DMA,
        ],
    )


def kernel(mk, mv, qk):
    mk_f = mk.reshape(CK, M)
    mv_f = mv.reshape(CV, M)
    qiT = jnp.swapaxes(qk.reshape(CK, HW), 0, 1) / math.sqrt(CK)
    qiT_p = jnp.pad(qiT, ((0, MROWS - HW), (0, 0)))
    at2, mvt, gm = _tc_call(qiT_p, mk_f, mv_f)
    out = _get_sc_call()(at2, mvt, gm)
    return jnp.swapaxes(out, 0, 1).reshape(1, CV, 30, 30)
